# baseline (speedup 1.0000x reference)
import jax
import jax.numpy as jnp
from jax import lax
from jax.experimental import pallas as pl
from jax.experimental.pallas import tpu as pltpu
from jax.experimental.pallas import tpu_sc as plsc

_B = 16384
_K = 100000
_D = 384

_NC = 2
_NS = 16
_NW = _NC * _NS
_BPW = _B // _NW
_C = 64
_NCHUNK = _BPW // _C
_NBUF = 5
_NUNIT = 3 * _NCHUNK


def _schedule():
    units = []
    for lvl in range(3):
        sizes = [_C] * _NCHUNK
        if lvl == 0:
            sizes = [_C // 2, _C // 2] + [_C] * (_NCHUNK - 1)
        elif lvl == 2:
            sizes = [_C] * (_NCHUNK - 1) + [_C // 2, _C // 2]
        roff = 0
        for sz in sizes:
            units.append((lvl, lvl * _D, roff, sz))
            roff += sz
        assert roff == _BPW
    return units


_UNITS = _schedule()
_NUNIT = len(_UNITS)


def _gather_body(*refs):
    t0, t1, t2, cb0, cb1, cb2, out = refs[:7]
    rbufs = refs[7:7 + _NBUF]
    gsems = refs[7 + _NBUF:7 + 2 * _NBUF]
    osems = refs[7 + 2 * _NBUF:7 + 3 * _NBUF]
    ixs = refs[7 + 3 * _NBUF:]
    cbs = (cb0, cb1, cb2)

    wid = lax.axis_index("s") * _NC + lax.axis_index("c")
    base = wid * _BPW

    pltpu.sync_copy(t0.at[pl.ds(base, _BPW)], ixs[0])

    def fire_gather(j):
        lvl, _, roff, sz = _UNITS[j]
        pltpu.async_copy(cbs[lvl].at[ixs[lvl].at[pl.ds(roff, sz)]],
                         rbufs[j % _NBUF].at[pl.ds(0, sz), :],
                         gsems[j % _NBUF])

    def out_copy(j):
        _, col, roff, sz = _UNITS[j]
        return pltpu.make_async_copy(
            rbufs[j % _NBUF].at[pl.ds(0, sz), :],
            out.at[pl.ds(base + roff, sz), pl.ds(col, _D)],
            osems[j % _NBUF])

    for j in range(_NBUF):
        fire_gather(j)
    pltpu.sync_copy(t1.at[pl.ds(base, _BPW)], ixs[1])
    pltpu.sync_copy(t2.at[pl.ds(base, _BPW)], ixs[2])
    for j in range(_NUNIT):
        lvl, _, roff, sz = _UNITS[j]
        b = j % _NBUF
        pltpu.make_async_copy(cbs[lvl].at[ixs[lvl].at[pl.ds(roff, sz)]],
                              rbufs[b].at[pl.ds(0, sz), :], gsems[b]).wait()
        out_copy(j).start()
        if j + _NBUF < _NUNIT:
            out_copy(j).wait()
            fire_gather(j + _NBUF)
    for j in range(_NUNIT - _NBUF, _NUNIT):
        out_copy(j).wait()


def kernel(tokens, codebook0, codebook1, codebook2):
    t0 = tokens[:, 0]
    t1 = tokens[:, 1]
    t2 = tokens[:, 2]
    mesh = plsc.VectorSubcoreMesh(core_axis_name="c", subcore_axis_name="s")
    scratch = (
        [pltpu.VMEM((_C, _D), jnp.float32)] * _NBUF
        + [pltpu.SemaphoreType.DMA] * (2 * _NBUF)
        + [pltpu.VMEM((_BPW,), jnp.int32)] * 3
    )
    run = pl.kernel(
        _gather_body,
        out_type=jax.ShapeDtypeStruct((_B, 3 * _D), jnp.float32),
        mesh=mesh,
        scratch_types=scratch,
        compiler_params=pltpu.CompilerParams(use_tc_tiling_on_sc=False),
    )
    return run(t0, t1, t2, codebook0, codebook1, codebook2)


# speedup vs baseline: 7.8853x; 7.8853x over previous
import jax
import jax.numpy as jnp
from jax import lax
from jax.experimental import pallas as pl
from jax.experimental.pallas import tpu as pltpu
from jax.experimental.pallas import tpu_sc as plsc

_B = 16384
_K = 100000
_D = 384

_NC = 2
_NS = 16
_NW = _NC * _NS
_BPW = _B // _NW
_C = 64
_NCHUNK = _BPW // _C
_NBUF = 5
_NUNIT = 3 * _NCHUNK


def _schedule():
    units = []
    for lvl in range(3):
        sizes = [_C] * _NCHUNK
        if lvl == 0:
            sizes = [_C // 2, _C // 2] + [_C] * (_NCHUNK - 1)
        elif lvl == 2:
            sizes = [_C] * (_NCHUNK - 1) + [_C // 2, _C // 2]
        roff = 0
        for sz in sizes:
            units.append((lvl, lvl * _D, roff, sz))
            roff += sz
        assert roff == _BPW
    return units


_UNITS = _schedule()
_NUNIT = len(_UNITS)


def _gather_body(*refs):
    t0, t1, t2, cb0, cb1, cb2, out = refs[:7]
    rbufs = refs[7:7 + _NBUF]
    gsems = refs[7 + _NBUF:7 + 2 * _NBUF]
    osems = refs[7 + 2 * _NBUF:7 + 3 * _NBUF]
    ixs = refs[7 + 3 * _NBUF:]
    cbs = (cb0, cb1, cb2)

    wid = lax.axis_index("s") * _NC + lax.axis_index("c")
    base = wid * _BPW

    pltpu.sync_copy(t0.at[pl.ds(base, _BPW)], ixs[0])

    def fire_gather(j):
        lvl, _, roff, sz = _UNITS[j]
        pltpu.async_copy(cbs[lvl].at[ixs[lvl].at[pl.ds(roff, sz)]],
                         rbufs[j % _NBUF].at[pl.ds(0, sz), :],
                         gsems[j % _NBUF])

    def out_copy(j):
        _, col, roff, sz = _UNITS[j]
        return pltpu.make_async_copy(
            rbufs[j % _NBUF].at[pl.ds(0, sz), :],
            out.at[pl.ds(base + roff, sz), pl.ds(col, _D)],
            osems[j % _NBUF])

    for j in range(_NBUF):
        fire_gather(j)
    pltpu.sync_copy(t1.at[pl.ds(base, _BPW)], ixs[1])
    pltpu.sync_copy(t2.at[pl.ds(base, _BPW)], ixs[2])
    for j in range(_NUNIT):
        lvl, _, roff, sz = _UNITS[j]
        b = j % _NBUF
        pltpu.make_async_copy(cbs[lvl].at[ixs[lvl].at[pl.ds(roff, sz)]],
                              rbufs[b].at[pl.ds(0, sz), :], gsems[b]).wait()
        out_copy(j).start()
        if j + _NBUF < _NUNIT:
            out_copy(j).wait()
            fire_gather(j + _NBUF)
    for j in range(_NUNIT - _NBUF, _NUNIT):
        out_copy(j).wait()


def kernel(tokens, codebook0, codebook1, codebook2):
    t0 = tokens[:, 0]
    t1 = tokens[:, 1]
    t2 = tokens[:, 2]
    mesh = plsc.VectorSubcoreMesh(core_axis_name="c", subcore_axis_name="s")
    scratch = (
        [pltpu.VMEM((_C, _D), jnp.float32)] * _NBUF
        + [pltpu.SemaphoreType.DMA] * (2 * _NBUF)
        + [pltpu.VMEM((_BPW,), jnp.int32)] * 3
    )
    run = pl.kernel(
        _gather_body,
        out_type=jax.ShapeDtypeStruct((_B, 3 * _D), jnp.float32),
        mesh=mesh,
        scratch_types=scratch,
    )
    return run(t0, t1, t2, codebook0, codebook1, codebook2)


# speedup vs baseline: 7.8943x; 1.0011x over previous
import jax
import jax.numpy as jnp
from jax import lax
from jax.experimental import pallas as pl
from jax.experimental.pallas import tpu as pltpu
from jax.experimental.pallas import tpu_sc as plsc

_B = 16384
_K = 100000
_D = 384

_NC = 2
_NS = 16
_NW = _NC * _NS
_BPW = _B // _NW
_C = 64
_NCHUNK = _BPW // _C
_NBUF = 5


def _schedule():
    units = []
    for lvl in range(3):
        sizes = [_C] * _NCHUNK
        if lvl == 0:
            sizes = [_C // 2, _C // 2] + [_C] * (_NCHUNK - 1)
        elif lvl == 2:
            sizes = [_C] * (_NCHUNK - 1) + [_C // 2, _C // 2]
        roff = 0
        for sz in sizes:
            units.append((lvl, lvl * _D, roff, sz))
            roff += sz
        assert roff == _BPW
    return units


_UNITS = _schedule()
_NUNIT = len(_UNITS)


def _gather_body(*refs):
    t0, t1, t2, cb0, cb1, cb2, out = refs[:7]
    rbufs = refs[7:7 + _NBUF]
    gsems = refs[7 + _NBUF:7 + 2 * _NBUF]
    osems = refs[7 + 2 * _NBUF:7 + 3 * _NBUF]
    ixs = refs[7 + 3 * _NBUF:]
    cbs = (cb0, cb1, cb2)

    wid = lax.axis_index("s") * _NC + lax.axis_index("c")
    base = wid * _BPW

    pltpu.sync_copy(t0.at[pl.ds(base, _BPW)], ixs[0])

    def fire_gather(j):
        lvl, _, roff, sz = _UNITS[j]
        pltpu.async_copy(cbs[lvl].at[ixs[lvl].at[pl.ds(roff, sz)]],
                         rbufs[j % _NBUF].at[pl.ds(0, sz), :],
                         gsems[j % _NBUF])

    def out_copy(j):
        _, col, roff, sz = _UNITS[j]
        return pltpu.make_async_copy(
            rbufs[j % _NBUF].at[pl.ds(0, sz), :],
            out.at[pl.ds(base + roff, sz), pl.ds(col, _D)],
            osems[j % _NBUF])

    for j in range(_NBUF):
        fire_gather(j)
    pltpu.sync_copy(t1.at[pl.ds(base, _BPW)], ixs[1])
    pltpu.sync_copy(t2.at[pl.ds(base, _BPW)], ixs[2])
    for j in range(_NUNIT):
        lvl, _, roff, sz = _UNITS[j]
        b = j % _NBUF
        pltpu.make_async_copy(cbs[lvl].at[ixs[lvl].at[pl.ds(roff, sz)]],
                              rbufs[b].at[pl.ds(0, sz), :], gsems[b]).wait()
        out_copy(j).start()
        if j + _NBUF < _NUNIT:
            out_copy(j).wait()
            fire_gather(j + _NBUF)
    for j in range(_NUNIT - _NBUF, _NUNIT):
        out_copy(j).wait()


def kernel(tokens, codebook0, codebook1, codebook2):
    t0 = tokens[:, 0]
    t1 = tokens[:, 1]
    t2 = tokens[:, 2]
    mesh = plsc.VectorSubcoreMesh(core_axis_name="c", subcore_axis_name="s")
    scratch = (
        [pltpu.VMEM((_C, _D), jnp.float32)] * _NBUF
        + [pltpu.SemaphoreType.DMA] * (2 * _NBUF)
        + [pltpu.VMEM((_BPW,), jnp.int32)] * 3
    )
    run = pl.kernel(
        _gather_body,
        out_type=jax.ShapeDtypeStruct((_B, 3 * _D), jnp.float32),
        mesh=mesh,
        scratch_types=scratch,
    )
    return run(t0, t1, t2, codebook0, codebook1, codebook2)
